# baseline (speedup 1.0000x reference)
import functools

import jax
import jax.numpy as jnp
from jax import lax
from jax.experimental import pallas as pl
from jax.experimental.pallas import tpu as pltpu
from jax.experimental.pallas import tpu_sc as plsc

L = 16


@functools.lru_cache(maxsize=None)
def _make_encoder(B, C, V, D):
    info = plsc.get_sparse_core_info()
    NC, NS = info.num_cores, info.num_subcores
    NW = NC * NS
    b_per_w = B // NW
    BC = 32
    NCH = b_per_w // BC
    ROWS = BC * C
    G = 104
    NG = ROWS // G
    KD = D // L
    W = 2 * D

    mesh = plsc.VectorSubcoreMesh(core_axis_name="c", subcore_axis_name="s")

    @functools.partial(
        pl.kernel,
        mesh=mesh,
        compiler_params=pltpu.CompilerParams(needs_layout_passes=False),
        out_type=jax.ShapeDtypeStruct((B, W), jnp.float32),
        scratch_types=[
            pltpu.VMEM((ROWS,), jnp.int32),
            pltpu.VMEM((ROWS,), jnp.int32),
            pltpu.VMEM((ROWS, W), jnp.float32),
            pltpu.VMEM((1, W), jnp.float32),
            pltpu.VMEM((BC, W), jnp.float32),
            pltpu.SemaphoreType.DMA,
        ],
    )
    def enc(ids_hbm, tab_hbm, out_hbm, idx_v, sidx_v, buf_v, r0_v, out_v,
            sem):
        wid = lax.axis_index("s") * NC + lax.axis_index("c")
        row_base = wid * b_per_w
        pltpu.sync_copy(tab_hbm.at[pl.ds(0, 1)], r0_v)
        r0 = [r0_v[0, k * L:(k + 1) * L] for k in range(KD)]
        lanes = lax.iota(jnp.int32, L)
        ovw = jnp.minimum(jnp.maximum(lanes - (2 * L - C - 1), 0), 1)

        def chunk_body(ch, carry):
            b0 = row_base + ch * BC
            pltpu.sync_copy(ids_hbm.at[pl.ds(b0 * C, ROWS)], idx_v)
            for i in range(ROWS // L):
                sidx_v[i * L:(i + 1) * L] = lax.shift_right_logical(
                    idx_v[i * L:(i + 1) * L], 1)
            cps = [
                pltpu.async_copy(
                    tab_hbm.at[sidx_v.at[pl.ds(g * G, G)]],
                    buf_v.at[pl.ds(g * G, G)], sem)
                for g in range(NG)
            ]
            for cp in cps:
                cp.wait()

            def b_body(b, carry2):
                r = b * C
                v0 = idx_v[pl.ds(r, L)]
                v1 = idx_v[pl.ds(r + C - L, L)]
                offs = [(v0[c] & 1) * D for c in range(L)] + \
                       [(v1[c - (C - L)] & 1) * D for c in range(L, C)]
                accs = [buf_v[r, pl.ds(offs[0] + k * L, L)]
                        for k in range(KD)]
                for c in range(1, C):
                    for k in range(KD):
                        accs[k] = accs[k] + buf_v[
                            r + c, pl.ds(offs[c] + k * L, L)]
                z0 = 1 - jnp.minimum(v0, 1)
                z1 = (1 - jnp.minimum(v1, 1)) * ovw
                n0 = jnp.sum(z0 + z1)
                n0v = jnp.full((L,), n0).astype(jnp.float32)
                inv = 1.0 / ((float(C) - n0v) + 1e-8)
                for k in range(KD):
                    out_v[b, k * L:(k + 1) * L] = (accs[k] - n0v * r0[k]) * inv
                return carry2

            lax.fori_loop(0, BC, b_body, 0)
            pltpu.sync_copy(out_v, out_hbm.at[pl.ds(b0, BC)])
            return carry

        lax.fori_loop(0, NCH, chunk_body, 0)

    return enc


def kernel(category_ids, embedding_weight):
    B, C = category_ids.shape
    V, D = embedding_weight.shape
    ids_flat = category_ids.reshape(-1).astype(jnp.int32)
    tab2 = (embedding_weight.T.reshape(D, V // 2, 2)
            .transpose(1, 2, 0).reshape(V // 2, 2 * D))
    out = _make_encoder(B, C, V, D)(ids_flat, tab2)
    return out[:, :D]


# speedup vs baseline: 1.3299x; 1.3299x over previous
import functools

import jax
import jax.numpy as jnp
from jax import lax
from jax.experimental import pallas as pl
from jax.experimental.pallas import tpu as pltpu
from jax.experimental.pallas import tpu_sc as plsc

L = 16


@functools.lru_cache(maxsize=None)
def _make_encoder(B, C, V, D):
    info = plsc.get_sparse_core_info()
    NC, NS = info.num_cores, info.num_subcores
    NW = NC * NS
    b_per_w = B // NW
    BC = 32
    NCH = b_per_w // BC
    ROWS = BC * C
    G = 104
    NG = ROWS // G
    KD = D // L

    mesh = plsc.VectorSubcoreMesh(core_axis_name="c", subcore_axis_name="s")

    @functools.partial(
        pl.kernel,
        mesh=mesh,
        compiler_params=pltpu.CompilerParams(
            use_tc_tiling_on_sc=False, needs_layout_passes=False),
        out_type=jax.ShapeDtypeStruct((B, D), jnp.float32),
        scratch_types=[
            pltpu.VMEM((ROWS,), jnp.int32),
            pltpu.VMEM((ROWS,), jnp.int32),
            pltpu.VMEM((ROWS, D), jnp.float32),
            pltpu.VMEM((ROWS, D), jnp.float32),
            pltpu.VMEM((1, D), jnp.float32),
            pltpu.VMEM((BC, D), jnp.float32),
            pltpu.SemaphoreType.DMA,
            pltpu.SemaphoreType.DMA,
            pltpu.SemaphoreType.DMA,
        ],
    )
    def enc(ids_hbm, tab_hbm, out_hbm, idx0, idx1, buf0, buf1, r0_v, out_v,
            g0sem, g1sem, isem):
        wid = lax.axis_index("s") * NC + lax.axis_index("c")
        row_base = wid * b_per_w
        pltpu.sync_copy(tab_hbm.at[pl.ds(0, 1)], r0_v)
        r0 = [r0_v[0, k * L:(k + 1) * L] for k in range(KD)]
        lanes = lax.iota(jnp.int32, L)
        ovw = jnp.minimum(jnp.maximum(lanes - (2 * L - C - 1), 0), 1)

        def start_gathers(idx_v, buf_v, sem):
            for g in range(NG):
                pltpu.async_copy(
                    tab_hbm.at[idx_v.at[pl.ds(g * G, G)]],
                    buf_v.at[pl.ds(g * G, G)], sem)

        def drain_gathers(buf_v, sem):
            pltpu.make_async_copy(tab_hbm.at[pl.ds(0, ROWS)], buf_v,
                                  sem).wait()

        def start_ids(ch, idx_v):
            pltpu.async_copy(
                ids_hbm.at[pl.ds((row_base + ch * BC) * C, ROWS)], idx_v,
                isem)

        def drain_ids(idx_v):
            pltpu.make_async_copy(ids_hbm.at[pl.ds(0, ROWS)], idx_v,
                                  isem).wait()

        def compute(ch, idx_v, buf_v):
            def b_body(b, carry2):
                r = b * C
                accs = [buf_v[r, k * L:(k + 1) * L] for k in range(KD)]
                for c in range(1, C):
                    for k in range(KD):
                        accs[k] = accs[k] + buf_v[r + c, k * L:(k + 1) * L]
                v0 = idx_v[pl.ds(r, L)]
                v1 = idx_v[pl.ds(r + C - L, L)]
                z0 = 1 - jnp.minimum(v0, 1)
                z1 = (1 - jnp.minimum(v1, 1)) * ovw
                n0 = jnp.sum(z0 + z1)
                n0v = jnp.full((L,), n0).astype(jnp.float32)
                inv = 1.0 / ((float(C) - n0v) + 1e-8)
                for k in range(KD):
                    out_v[b, k * L:(k + 1) * L] = (accs[k] - n0v * r0[k]) * inv
                return carry2

            lax.fori_loop(0, BC, b_body, 0)
            pltpu.sync_copy(out_v, out_hbm.at[pl.ds(row_base + ch * BC, BC)])

        pltpu.sync_copy(ids_hbm.at[pl.ds(row_base * C, ROWS)], idx0)
        start_gathers(idx0, buf0, g0sem)
        start_ids(1, idx1)

        def pair_body(h, carry):
            ch0 = 2 * h
            drain_ids(idx1)
            start_gathers(idx1, buf1, g1sem)
            drain_gathers(buf0, g0sem)
            compute(ch0, idx0, buf0)
            start_ids(ch0 + 2, idx0)
            drain_ids(idx0)
            start_gathers(idx0, buf0, g0sem)
            drain_gathers(buf1, g1sem)
            compute(ch0 + 1, idx1, buf1)
            start_ids(ch0 + 3, idx1)
            return carry

        lax.fori_loop(0, NCH // 2 - 1, pair_body, 0)

        drain_ids(idx1)
        start_gathers(idx1, buf1, g1sem)
        drain_gathers(buf0, g0sem)
        compute(NCH - 2, idx0, buf0)
        drain_gathers(buf1, g1sem)
        compute(NCH - 1, idx1, buf1)

    return enc


def kernel(category_ids, embedding_weight):
    B, C = category_ids.shape
    V, D = embedding_weight.shape
    ids_flat = category_ids.reshape(-1).astype(jnp.int32)
    return _make_encoder(B, C, V, D)(ids_flat, embedding_weight)
